# baseline (speedup 1.0000x reference)
import jax
import jax.numpy as jnp
from jax import lax
from jax.experimental import pallas as pl
from jax.experimental.pallas import tpu as pltpu

_CHUNK = 1000
_N = 10000
_NCHUNKS = _N // _CHUNK
_NSTREAMS = 5


def _mlp_kernel(x0, x1, x2, x3, x4, w0t_ref, b0_ref, w1_ref, b1_ref, w2r_ref,
                b2_ref, out_ref, buf0, buf1, buf2, buf3, buf4, sem):
    xs = (x0, x1, x2, x3, x4)
    bufs = (buf0, buf1, buf2, buf3, buf4)

    def copy(c):
        j = c % _NSTREAMS
        s = (c // _NSTREAMS) % 2
        return pltpu.make_async_copy(
            xs[j].at[pl.ds(c * _CHUNK, _CHUNK), :], bufs[j].at[s],
            sem.at[j, s])

    for c in range(_NSTREAMS):
        copy(c).start()

    w1 = w1_ref[...]
    w01t = lax.dot_general(w1, w0t_ref[...], (((0,), (0,)), ((), ())),
                           preferred_element_type=jnp.float32)
    b01t = lax.dot_general(w1, b0_ref[...], (((0,), (1,)), ((), ())),
                           preferred_element_type=jnp.float32) + b1_ref[...].T
    w2r = w2r_ref[...]

    outs = []
    for c in range(_NCHUNKS):
        copy(c).wait()
        xb = bufs[c % _NSTREAMS][(c // _NSTREAMS) % 2]
        if c + _NSTREAMS < _NCHUNKS:
            copy(c + _NSTREAMS).start()
        h_t = lax.dot_general(w01t, xb, (((1,), (1,)), ((), ())),
                              preferred_element_type=jnp.float32)
        h_t = jnp.maximum(h_t + b01t, 0.0)
        outs.append(lax.dot_general(w2r, h_t, (((1,), (0,)), ((), ())),
                                    preferred_element_type=jnp.float32))
    out_ref[...] = jnp.concatenate(outs, axis=1) + b2_ref[...]


def kernel(x, edge_index, W0, b0, W1, b1, W2, b2):
    del edge_index
    n, d = x.shape
    hid = W0.shape[1]
    end_hid = W1.shape[1]
    out_dim = W2.shape[1]
    out = pl.pallas_call(
        _mlp_kernel,
        in_specs=[pl.BlockSpec(memory_space=pl.ANY)] * _NSTREAMS + [
            pl.BlockSpec((hid, d), lambda: (0, 0)),
            pl.BlockSpec((1, hid), lambda: (0, 0)),
            pl.BlockSpec((hid, end_hid), lambda: (0, 0)),
            pl.BlockSpec((1, end_hid), lambda: (0, 0)),
            pl.BlockSpec((1, end_hid), lambda: (0, 0)),
            pl.BlockSpec((1, out_dim), lambda: (0, 0)),
        ],
        out_specs=pl.BlockSpec((1, n), lambda: (0, 0)),
        out_shape=jax.ShapeDtypeStruct((1, n), jnp.float32),
        scratch_shapes=[pltpu.VMEM((2, _CHUNK, 128), jnp.float32)
                        for _ in range(_NSTREAMS)] +
                       [pltpu.SemaphoreType.DMA((_NSTREAMS, 2))],
    )(x, x, x, x, x, W0.T, b0.reshape(1, hid), W1, b1.reshape(1, end_hid),
      W2.reshape(1, end_hid), b2.reshape(1, out_dim))
    return out.reshape(n, out_dim)
